# baseline (speedup 1.0000x reference)
import jax
import jax.numpy as jnp
from jax import lax
from jax.experimental import pallas as pl
from jax.experimental.pallas import tpu as pltpu
from jax.experimental.pallas import tpu_sc as plsc

NODE_NUM = 100000
N = 50000
E = 800000
NUM_GRAPHS = 64
GLOVE = 100
DIM = 100
NUM_CLASS = 52

NC, NS, LANES = 2, 16, 16
NW = NC * NS
N_PAD = 50176
E_PAD = 802816
NPW = N_PAD // NW
EPW = E_PAD // NW
EPT = E_PAD // NS
BLK = 128
F = 128
CH = 32
ACC2_ROWS = 72

_mesh = plsc.VectorSubcoreMesh(
    core_axis_name="c", subcore_axis_name="s", num_cores=NC, num_subcores=NS)
_params = pltpu.CompilerParams(
    use_tc_tiling_on_sc=False, needs_layout_passes=False)

_f32 = jnp.float32
_i32 = jnp.int32


def _k0_body(emb_ref, et_ref, out_ref):
    x = emb_ref[...] * et_ref[...]
    out_ref[...] = jnp.concatenate(
        [x, jnp.zeros((x.shape[0], F - GLOVE), _f32)], axis=1)


def _run_k0(embedding, etans):
    blk = 1000
    return pl.pallas_call(
        _k0_body,
        grid=(NODE_NUM // blk,),
        in_specs=[
            pl.BlockSpec((blk, GLOVE), lambda i: (i, 0)),
            pl.BlockSpec((blk, 1), lambda i: (i, 0)),
        ],
        out_specs=pl.BlockSpec((blk, F), lambda i: (i, 0)),
        out_shape=jax.ShapeDtypeStruct((NODE_NUM, F), _f32),
    )(embedding, etans.reshape(NODE_NUM, 1))


def _mul_rows(buf, wref, krow, g, nv):
    w16 = wref[krow, pl.ds(g * 16, 16)]
    dn = lax.GatherDimensionNumbers(
        offset_dims=(), collapsed_slice_dims=(0,), start_index_map=(0,))
    for j in range(16):
        e = g * 16 + j
        ws = lax.gather(w16, jnp.full((16, 1), j, _i32), dn, (1,),
                        mode=lax.GatherScatterMode.PROMISE_IN_BOUNDS)
        for v in range(nv):
            buf[e, pl.ds(16 * v, 16)] = buf[e, pl.ds(16 * v, 16)] * ws


def _k2_body(emb2, node_p, src2, dst2, ea2, ean,
             h0, h1, h2, h3, w0, w1, agg0, agg1, agg2, agg3,
             sidx4, didx4, wv4, bufa, bufb, sbufa, sbufb, cbuf, cbt, zbuf,
             acc, semga, semgb, semsa, semsb):
    core = lax.axis_index("c")
    s = lax.axis_index("s")
    hrefs = (h0, h1, h2, h3)
    arefs = (agg0, agg1, agg2, agg3)
    wrefs = (w0, w1)
    bufs = (bufa, bufb)
    semG = (semga, semgb)
    semS = (semsa, semsb)

    def zr(k, carry):
        zbuf[k, pl.ds(0, 16)] = jnp.zeros((16,), _f32)
        zbuf[k, pl.ds(16, 16)] = jnp.zeros((16,), _f32)
        return carry

    lax.fori_loop(0, 64, zr, 0)
    zoff = s * (N_PAD // NS)
    rbase = s * (EPT // BLK)
    NPT = N_PAD // NS

    for c_id in range(NC):
        @pl.when(core == c_id)
        def _phase0(c_id=c_id):
            def node_block(base, nb):
                pltpu.sync_copy(node_p.at[pl.ds(base, nb)],
                                sidx4.at[0].at[pl.ds(0, nb)])
                pltpu.async_copy(
                    emb2.at[sidx4.at[0].at[pl.ds(0, nb)]],
                    cbuf.at[pl.ds(0, nb)], semG[0]).wait()
                for t in range(2):
                    chunk = 2 * c_id + t

                    def cp(e, carry, chunk=chunk):
                        a = cbuf[e, pl.ds(32 * chunk, 16)]
                        b = cbuf[e, pl.ds(32 * chunk + 16, 16)]
                        cbt[e, :] = plsc.pack(
                            a, b, format=plsc.PackFormat.INTERLEAVED)
                        return carry

                    lax.fori_loop(0, nb, cp, 0)
                    pltpu.sync_copy(cbt.at[pl.ds(0, nb)],
                                    hrefs[chunk].at[pl.ds(base, nb)])

            def blockA(i, carry):
                node_block(s * NPT + i * 64, 64)
                return carry

            lax.fori_loop(0, NPT // 64, blockA, 0)

            def blockW(i, carry):
                row = rbase + i * 4
                pltpu.sync_copy(ea2.at[pl.ds(row, 4)], didx4)
                descs = [pltpu.async_copy(ean.at[didx4.at[k]],
                                          wv4.at[k], semG[0])
                         for k in range(4)]
                for d in descs:
                    d.wait()
                pltpu.sync_copy(wv4, wrefs[c_id].at[pl.ds(row, 4)])
                return carry

            lax.fori_loop(0, EPT // BLK // 4, blockW, 0)

    for ci in range(2):
        def zcp(k, carry):
            pltpu.sync_copy(zbuf, acc.at[pl.ds(zoff + k * 64, 64)])
            return carry

        lax.fori_loop(0, (N_PAD // NS) // 64, zcp, 0)
        plsc.subcore_barrier()

        for c_id in range(NC):
            chunk = c_id * 2 + ci

            @pl.when(core == c_id)
            def _scatter(chunk=chunk, c_id=c_id):
                hdummy = hrefs[chunk].at[pl.ds(0, BLK)]
                sdummy = arefs[chunk].at[pl.ds(0, BLK)]
                sbufs = (sbufa, sbufb)

                def sb(i, carry):
                    row = rbase + i * 4
                    pltpu.sync_copy(src2.at[pl.ds(row, 4)], sidx4)
                    pltpu.sync_copy(dst2.at[pl.ds(row, 4)], didx4)
                    pltpu.sync_copy(wrefs[c_id].at[pl.ds(row, 4)], wv4)
                    pltpu.async_copy(hrefs[chunk].at[sidx4.at[0]],
                                     bufs[0], semG[0])
                    for k in range(4):
                        if k < 3:
                            pltpu.async_copy(
                                hrefs[chunk].at[sidx4.at[k + 1]],
                                bufs[(k + 1) % 2], semG[(k + 1) % 2])
                        pltpu.make_async_copy(hdummy, bufs[k % 2],
                                              semG[k % 2]).wait()
                        if k < 2:
                            @pl.when(i > 0)
                            def _drainS(k=k):
                                pltpu.make_async_copy(
                                    sdummy, sbufs[k % 2], semS[k % 2]).wait()
                        else:
                            pltpu.make_async_copy(
                                sdummy, sbufs[k % 2], semS[k % 2]).wait()

                        def mg(g, c2, k=k):
                            w16 = wv4[k, pl.ds(g * 16, 16)]
                            dn = lax.GatherDimensionNumbers(
                                offset_dims=(), collapsed_slice_dims=(0,),
                                start_index_map=(0,))
                            for j in range(16):
                                e = g * 16 + j
                                ws = lax.gather(
                                    w16, jnp.full((16, 1), j, _i32), dn,
                                    (1,),
                                    mode=lax.GatherScatterMode
                                    .PROMISE_IN_BOUNDS)
                                a, b = plsc.unpack(
                                    bufs[k % 2][e, :],
                                    format=plsc.PackFormat.INTERLEAVED)
                                sbufs[k % 2][e, pl.ds(0, 16)] = a * ws
                                sbufs[k % 2][e, pl.ds(16, 16)] = b * ws
                            return c2

                        lax.fori_loop(0, 8, mg, 0)
                        pltpu.async_copy(sbufs[k % 2], acc.at[didx4.at[k]],
                                         semS[k % 2], add=True)
                    return carry

                lax.fori_loop(0, EPT // BLK // 4, sb, 0)
                pltpu.make_async_copy(sdummy, sbufs[0], semS[0]).wait()
                pltpu.make_async_copy(sdummy, sbufs[1], semS[1]).wait()

        plsc.subcore_barrier()

        for c_id in range(NC):
            chunk = c_id * 2 + ci

            @pl.when(core == c_id)
            def _writeback(chunk=chunk):
                def wb(k, carry):
                    off = zoff + k * 64
                    pltpu.sync_copy(acc.at[pl.ds(off, 64)],
                                    arefs[chunk].at[pl.ds(off, 64)])
                    return carry

                lax.fori_loop(0, (N_PAD // NS) // 64, wb, 0)

        plsc.subcore_barrier()


def _run_k2(emb2, node_p, src2, dst2, ea2, ean):
    _bf16 = jnp.bfloat16
    out_type = ([jax.ShapeDtypeStruct((N_PAD, CH), _bf16)] * 4
                + [jax.ShapeDtypeStruct((E_PAD // BLK, BLK), _f32)] * 2
                + [jax.ShapeDtypeStruct((N_PAD, CH), _f32)] * 4)
    k = pl.kernel(
        _k2_body,
        out_type=out_type,
        mesh=_mesh,
        compiler_params=_params,
        scratch_types=[
            pltpu.VMEM((4, BLK), _i32),
            pltpu.VMEM((4, BLK), _i32),
            pltpu.VMEM((4, BLK), _f32),
            pltpu.VMEM((BLK, CH), _bf16),
            pltpu.VMEM((BLK, CH), _bf16),
            pltpu.VMEM((BLK, CH), _f32),
            pltpu.VMEM((BLK, CH), _f32),
            pltpu.VMEM((64, F), _f32),
            pltpu.VMEM((64, CH), _bf16),
            pltpu.VMEM((64, CH), _f32),
            pltpu.VMEM_SHARED((N_PAD, CH), _f32),
            pltpu.SemaphoreType.DMA,
            pltpu.SemaphoreType.DMA,
            pltpu.SemaphoreType.DMA,
            pltpu.SemaphoreType.DMA,
        ],
    )
    outs = k(emb2, node_p, src2, dst2, ea2, ean)
    return outs[6:10], outs[4]


def _k3_body(a0, a1, a2, a3, m0, m1, m2, m3, b1_ref, out_ref):
    y = jnp.dot(a0[...], m0[...], preferred_element_type=_f32)
    y += jnp.dot(a1[...], m1[...], preferred_element_type=_f32)
    y += jnp.dot(a2[...], m2[...], preferred_element_type=_f32)
    y += jnp.dot(a3[...], m3[...], preferred_element_type=_f32)
    out_ref[...] = jnp.maximum(y + b1_ref[...], 0.0)


def _run_k3(aggs_packed, Ms, b1t):
    blk = 448
    grid = (N_PAD // 4) // blk
    y = pl.pallas_call(
        _k3_body,
        grid=(grid,),
        in_specs=[pl.BlockSpec((blk, F), lambda i: (i, 0))] * 4
        + [pl.BlockSpec((F, 4 * F), lambda i: (0, 0))] * 4
        + [pl.BlockSpec((1, 4 * F), lambda i: (0, 0))],
        out_specs=pl.BlockSpec((blk, 4 * F), lambda i: (i, 0)),
        out_shape=jax.ShapeDtypeStruct((N_PAD // 4, 4 * F), _f32),
    )(*aggs_packed, *Ms, b1t)
    return y.reshape(N_PAD, F)


def _k4_body(h1_hbm, src2, dst2, w2, batch_p,
             out2, outc,
             sidx4, didx4, gv4, wv4, bufa, bufb, zbuf2, obuf, gvn, gvn_t,
             idv, accs, acc2, accc, semga, semgb, semsa, semsb, semi):
    core = lax.axis_index("c")
    s = lax.axis_index("s")
    wid = s * NC + core
    bufs = (bufa, bufb)
    semG = (semga, semgb)
    semS = (semsa, semsb)
    hdummy = h1_hbm.at[pl.ds(0, BLK)]

    def zr(k, carry):
        for j in range(F // 16):
            obuf[k, pl.ds(16 * j, 16)] = jnp.full((16,), 1.0, _f32)
        return carry

    lax.fori_loop(0, BLK, zr, 0)

    def zr2(k, carry):
        for j in range(F // 16):
            zbuf2[k, pl.ds(16 * j, 16)] = jnp.zeros((16,), _f32)
        return carry

    lax.fori_loop(0, 8, zr2, 0)

    slab = s * ACC2_ROWS

    def zr3(k, carry):
        pltpu.sync_copy(zbuf2, accs.at[pl.ds(slab + k * 8, 8)])
        return carry

    lax.fori_loop(0, ACC2_ROWS // 8, zr3, 0)
    base16 = lax.iota(_i32, 16)
    for q in range(NUM_GRAPHS // 16):
        idv[pl.ds(16 * q, 16)] = base16 + 16 * q

    @pl.when(s == 0)
    def _zero_acc():
        def zcp(k, carry):
            pltpu.sync_copy(zbuf2, acc2.at[pl.ds(k * 8, 8)])
            return carry

        lax.fori_loop(0, ACC2_ROWS // 8, zcp, 0)

        def zcc(k, carry):
            pltpu.sync_copy(zbuf2, accc.at[pl.ds(k * 8, 8)])
            return carry

        lax.fori_loop(0, ACC2_ROWS // 8, zcc, 0)

    plsc.subcore_barrier()

    rbase = wid * (EPW // BLK)

    def eb(i, carry):
        row = rbase + i * 4
        pltpu.sync_copy(src2.at[pl.ds(row, 4)], sidx4)
        pltpu.sync_copy(dst2.at[pl.ds(row, 4)], didx4)
        pltpu.sync_copy(w2.at[pl.ds(row, 4)], wv4)
        for k in range(4):
            pltpu.async_copy(batch_p.at[didx4.at[k]], gv4.at[k], semi)

        @pl.when(i > 0)
        def _drain0():
            pltpu.make_async_copy(hdummy, bufs[0], semS[0]).wait()

        pltpu.async_copy(h1_hbm.at[sidx4.at[0]], bufs[0], semG[0])
        pltpu.make_async_copy(src2.at[pl.ds(0, 4)], gv4, semi).wait()
        soff = jnp.full((16,), slab, _i32)
        for k in range(4):
            for v in range(BLK // 16):
                gv4[k, pl.ds(16 * v, 16)] = gv4[k, pl.ds(16 * v, 16)] + soff
        for k in range(4):
            if k < 3:
                nb = (k + 1) % 2
                if k == 0:
                    @pl.when(i > 0)
                    def _drain1():
                        pltpu.make_async_copy(hdummy, bufs[1],
                                              semS[1]).wait()
                else:
                    pltpu.make_async_copy(hdummy, bufs[nb], semS[nb]).wait()
                pltpu.async_copy(h1_hbm.at[sidx4.at[k + 1]], bufs[nb],
                                 semG[nb])
            pltpu.make_async_copy(hdummy, bufs[k % 2], semG[k % 2]).wait()

            def mg(g, c2, k=k):
                _mul_rows(bufs[k % 2], wv4, k, g, F // 16)
                return c2

            lax.fori_loop(0, 8, mg, 0)
            pltpu.async_copy(bufs[k % 2], accs.at[gv4.at[k]], semS[k % 2],
                             add=True)
        return carry

    lax.fori_loop(0, EPW // BLK // 4, eb, 0)
    pltpu.make_async_copy(hdummy, bufs[0], semS[0]).wait()
    pltpu.make_async_copy(hdummy, bufs[1], semS[1]).wait()

    pltpu.sync_copy(accs.at[pl.ds(slab, NUM_GRAPHS)],
                    bufa.at[pl.ds(0, NUM_GRAPHS)])
    pltpu.sync_copy(bufa.at[pl.ds(0, NUM_GRAPHS)], acc2.at[idv], add=True)

    nbase = wid * NPW

    def cb(i, carry):
        pltpu.sync_copy(batch_p.at[pl.ds(nbase + i * BLK, BLK)], gvn)
        pltpu.sync_copy(obuf, accc.at[gvn], add=True)
        return carry

    lax.fori_loop(0, NPW // BLK, cb, 0)
    tb = nbase + (NPW // BLK) * BLK
    pltpu.sync_copy(batch_p.at[pl.ds(tb, NPW % BLK)], gvn_t)
    pltpu.sync_copy(obuf.at[pl.ds(0, NPW % BLK)], accc.at[gvn_t], add=True)

    plsc.subcore_barrier()

    @pl.when(s == 0)
    def _writeback():
        pltpu.sync_copy(acc2.at[pl.ds(0, NUM_GRAPHS)], out2.at[core])
        pltpu.sync_copy(accc.at[pl.ds(0, NUM_GRAPHS)], outc.at[core])


def _run_k4(h1, src2, dst2, w2, batch_p):
    k = pl.kernel(
        _k4_body,
        out_type=[jax.ShapeDtypeStruct((NC, NUM_GRAPHS, F), _f32),
                  jax.ShapeDtypeStruct((NC, NUM_GRAPHS, F), _f32)],
        mesh=_mesh,
        compiler_params=_params,
        scratch_types=[
            pltpu.VMEM((4, BLK), _i32),
            pltpu.VMEM((4, BLK), _i32),
            pltpu.VMEM((4, BLK), _i32),
            pltpu.VMEM((4, BLK), _f32),
            pltpu.VMEM((BLK, F), _f32),
            pltpu.VMEM((BLK, F), _f32),
            pltpu.VMEM((8, F), _f32),
            pltpu.VMEM((BLK, F), _f32),
            pltpu.VMEM((BLK,), _i32),
            pltpu.VMEM((NPW % BLK,), _i32),
            pltpu.VMEM((NUM_GRAPHS,), _i32),
            pltpu.VMEM_SHARED((NS * ACC2_ROWS, F), _f32),
            pltpu.VMEM_SHARED((ACC2_ROWS, F), _f32),
            pltpu.VMEM_SHARED((ACC2_ROWS, F), _f32),
            pltpu.SemaphoreType.DMA,
            pltpu.SemaphoreType.DMA,
            pltpu.SemaphoreType.DMA,
            pltpu.SemaphoreType.DMA,
            pltpu.SemaphoreType.DMA,
        ],
    )
    return k(h1, src2, dst2, w2, batch_p)


def _k5_body(o2_ref, oc_ref, w2_ref, b2_ref, out_ref):
    summed = o2_ref[0] + o2_ref[1]
    cnt = oc_ref[0] + oc_ref[1]
    cnt1 = jnp.maximum(cnt[:, 0:1], 1.0)
    pooled = summed * (1.0 / cnt1)
    logits = (jnp.dot(pooled[:, :DIM], w2_ref[...],
                      preferred_element_type=_f32) + b2_ref[...])
    out_ref[...] = logits


def _run_k5(out2, outc, W2, b2):
    return pl.pallas_call(
        _k5_body,
        out_shape=jax.ShapeDtypeStruct((NUM_GRAPHS, NUM_CLASS), _f32),
    )(out2, outc, W2, b2.reshape(1, NUM_CLASS))


def kernel(node, adj, edge_attr, batch, embedding, ean, etans, W1, b1, W2,
           b2):
    node = node.astype(_i32)
    adj = adj.astype(_i32)
    edge_attr = edge_attr.astype(_i32)
    batch = batch.astype(_i32)

    epad = E_PAD - E
    npad = N_PAD - N
    src2 = jnp.concatenate([adj[0], jnp.zeros((epad,), _i32)]).reshape(
        E_PAD // BLK, BLK)
    dst2 = jnp.concatenate([adj[1], jnp.full((epad,), N, _i32)]).reshape(
        E_PAD // BLK, BLK)
    ea2 = jnp.concatenate([edge_attr, jnp.zeros((epad,), _i32)]).reshape(
        E_PAD // BLK, BLK)
    node_p = jnp.concatenate([node, jnp.zeros((npad,), _i32)])
    batch_p = jnp.concatenate([batch, jnp.full((npad,), NUM_GRAPHS, _i32)])
    W1p = jnp.pad(W1, ((0, F - GLOVE), (0, F - DIM)))
    eye4 = jnp.eye(4, dtype=_f32)
    Ms = [jnp.kron(eye4, W1p[32 * c:32 * c + 32, :]) for c in range(4)]
    b1t = jnp.tile(jnp.pad(b1, (0, F - DIM)), 4).reshape(1, 4 * F)

    emb2 = _run_k0(embedding, etans)
    aggs, w2a = _run_k2(emb2, node_p, src2, dst2, ea2, ean)
    aggs_packed = [a.reshape(N_PAD // 4, F) for a in aggs]
    h1 = _run_k3(aggs_packed, Ms, b1t)
    out2, outc = _run_k4(h1, src2, dst2, w2a, batch_p)
    return _run_k5(out2, outc, W2, b2)


# speedup vs baseline: 1.0858x; 1.0858x over previous
import jax
import jax.numpy as jnp
from jax import lax
from jax.experimental import pallas as pl
from jax.experimental.pallas import tpu as pltpu
from jax.experimental.pallas import tpu_sc as plsc

NODE_NUM = 100000
N = 50000
E = 800000
NUM_GRAPHS = 64
GLOVE = 100
DIM = 100
NUM_CLASS = 52

NC, NS, LANES = 2, 16, 16
NW = NC * NS
N_PAD = 50176
E_PAD = 802816
NPW = N_PAD // NW
EPW = E_PAD // NW
EPT = E_PAD // NS
BLK = 128
F = 128
CH = 32
ACC2_ROWS = 72

_mesh = plsc.VectorSubcoreMesh(
    core_axis_name="c", subcore_axis_name="s", num_cores=NC, num_subcores=NS)
_params = pltpu.CompilerParams(
    use_tc_tiling_on_sc=False, needs_layout_passes=False)

_f32 = jnp.float32
_i32 = jnp.int32


def _k0_body(emb_ref, et_ref, out_ref):
    x = emb_ref[...] * et_ref[...]
    out_ref[...] = jnp.concatenate(
        [x, jnp.zeros((x.shape[0], F - GLOVE), _f32)], axis=1)


def _run_k0(embedding, etans):
    blk = 1000
    return pl.pallas_call(
        _k0_body,
        grid=(NODE_NUM // blk,),
        in_specs=[
            pl.BlockSpec((blk, GLOVE), lambda i: (i, 0)),
            pl.BlockSpec((blk, 1), lambda i: (i, 0)),
        ],
        out_specs=pl.BlockSpec((blk, F), lambda i: (i, 0)),
        out_shape=jax.ShapeDtypeStruct((NODE_NUM, F), _f32),
    )(embedding, etans.reshape(NODE_NUM, 1))


def _mul_rows(buf, wref, krow, g, nv):
    w16 = wref[krow, pl.ds(g * 16, 16)]
    dn = lax.GatherDimensionNumbers(
        offset_dims=(), collapsed_slice_dims=(0,), start_index_map=(0,))
    for j in range(16):
        e = g * 16 + j
        ws = lax.gather(w16, jnp.full((16, 1), j, _i32), dn, (1,),
                        mode=lax.GatherScatterMode.PROMISE_IN_BOUNDS)
        for v in range(nv):
            buf[e, pl.ds(16 * v, 16)] = buf[e, pl.ds(16 * v, 16)] * ws


def _k2_body(emb2, node_p, src2, dst2, ea2, ean,
             h0, h1, h2, h3, w0, w1, agg0, agg1, agg2, agg3,
             sidx4, didx4, wv4, bufa, bufb, cbuf, cbt, zbuf, acc,
             semga, semgb, semsa, semsb):
    core = lax.axis_index("c")
    s = lax.axis_index("s")
    hrefs = (h0, h1, h2, h3)
    arefs = (agg0, agg1, agg2, agg3)
    wrefs = (w0, w1)
    bufs = (bufa, bufb)
    semG = (semga, semgb)
    semS = (semsa, semsb)

    def zr(k, carry):
        zbuf[k, pl.ds(0, 16)] = jnp.zeros((16,), _f32)
        zbuf[k, pl.ds(16, 16)] = jnp.zeros((16,), _f32)
        return carry

    lax.fori_loop(0, 64, zr, 0)
    zoff = s * (N_PAD // NS)
    rbase = s * (EPT // BLK)
    NPT = N_PAD // NS

    for c_id in range(NC):
        @pl.when(core == c_id)
        def _phase0(c_id=c_id):
            def node_block(base, nb):
                pltpu.sync_copy(node_p.at[pl.ds(base, nb)],
                                sidx4.at[0].at[pl.ds(0, nb)])
                pltpu.async_copy(
                    emb2.at[sidx4.at[0].at[pl.ds(0, nb)]],
                    cbuf.at[pl.ds(0, nb)], semG[0]).wait()
                for t in range(2):
                    chunk = 2 * c_id + t

                    def cp(e, carry, chunk=chunk):
                        cbt[e, pl.ds(0, 16)] = \
                            cbuf[e, pl.ds(32 * chunk, 16)]
                        cbt[e, pl.ds(16, 16)] = \
                            cbuf[e, pl.ds(32 * chunk + 16, 16)]
                        return carry

                    lax.fori_loop(0, nb, cp, 0)
                    pltpu.sync_copy(cbt.at[pl.ds(0, nb)],
                                    hrefs[chunk].at[pl.ds(base, nb)])

            def blockA(i, carry):
                node_block(s * NPT + i * 64, 64)
                return carry

            lax.fori_loop(0, NPT // 64, blockA, 0)

            def blockW(i, carry):
                row = rbase + i * 4
                pltpu.sync_copy(ea2.at[pl.ds(row, 4)], didx4)
                descs = [pltpu.async_copy(ean.at[didx4.at[k]],
                                          wv4.at[k], semG[0])
                         for k in range(4)]
                for d in descs:
                    d.wait()
                pltpu.sync_copy(wv4, wrefs[c_id].at[pl.ds(row, 4)])
                return carry

            lax.fori_loop(0, EPT // BLK // 4, blockW, 0)

    for ci in range(2):
        def zcp(k, carry):
            pltpu.sync_copy(zbuf, acc.at[pl.ds(zoff + k * 64, 64)])
            return carry

        lax.fori_loop(0, (N_PAD // NS) // 64, zcp, 0)
        plsc.subcore_barrier()

        for c_id in range(NC):
            chunk = c_id * 2 + ci

            @pl.when(core == c_id)
            def _scatter(chunk=chunk, c_id=c_id):
                hdummy = hrefs[chunk].at[pl.ds(0, BLK)]

                def sb(i, carry):
                    row = rbase + i * 4
                    pltpu.sync_copy(src2.at[pl.ds(row, 4)], sidx4)
                    pltpu.sync_copy(dst2.at[pl.ds(row, 4)], didx4)
                    pltpu.sync_copy(wrefs[c_id].at[pl.ds(row, 4)], wv4)

                    @pl.when(i > 0)
                    def _drain0():
                        pltpu.make_async_copy(hdummy, bufs[0],
                                              semS[0]).wait()

                    pltpu.async_copy(hrefs[chunk].at[sidx4.at[0]],
                                     bufs[0], semG[0])
                    for k in range(4):
                        if k < 3:
                            nb = (k + 1) % 2
                            if k == 0:
                                @pl.when(i > 0)
                                def _drain1():
                                    pltpu.make_async_copy(
                                        hdummy, bufs[1], semS[1]).wait()
                            else:
                                pltpu.make_async_copy(
                                    hdummy, bufs[nb], semS[nb]).wait()
                            pltpu.async_copy(
                                hrefs[chunk].at[sidx4.at[k + 1]],
                                bufs[nb], semG[nb])
                        pltpu.make_async_copy(hdummy, bufs[k % 2],
                                              semG[k % 2]).wait()

                        def mg(g, c2, k=k):
                            _mul_rows(bufs[k % 2], wv4, k, g, 2)
                            return c2

                        lax.fori_loop(0, 8, mg, 0)
                        pltpu.async_copy(bufs[k % 2], acc.at[didx4.at[k]],
                                         semS[k % 2], add=True)
                    return carry

                lax.fori_loop(0, EPT // BLK // 4, sb, 0)
                pltpu.make_async_copy(hdummy, bufs[0], semS[0]).wait()
                pltpu.make_async_copy(hdummy, bufs[1], semS[1]).wait()

        plsc.subcore_barrier()

        for c_id in range(NC):
            chunk = c_id * 2 + ci

            @pl.when(core == c_id)
            def _writeback(chunk=chunk):
                def wb(k, carry):
                    off = zoff + k * 64
                    pltpu.sync_copy(acc.at[pl.ds(off, 64)],
                                    arefs[chunk].at[pl.ds(off, 64)])
                    return carry

                lax.fori_loop(0, (N_PAD // NS) // 64, wb, 0)

        plsc.subcore_barrier()


def _run_k2(emb2, node_p, src2, dst2, ea2, ean):
    out_type = ([jax.ShapeDtypeStruct((N_PAD, CH), _f32)] * 4
                + [jax.ShapeDtypeStruct((E_PAD // BLK, BLK), _f32)] * 2
                + [jax.ShapeDtypeStruct((N_PAD, CH), _f32)] * 4)
    k = pl.kernel(
        _k2_body,
        out_type=out_type,
        mesh=_mesh,
        compiler_params=_params,
        scratch_types=[
            pltpu.VMEM((4, BLK), _i32),
            pltpu.VMEM((4, BLK), _i32),
            pltpu.VMEM((4, BLK), _f32),
            pltpu.VMEM((BLK, CH), _f32),
            pltpu.VMEM((BLK, CH), _f32),
            pltpu.VMEM((64, F), _f32),
            pltpu.VMEM((64, CH), _f32),
            pltpu.VMEM((64, CH), _f32),
            pltpu.VMEM_SHARED((N_PAD, CH), _f32),
            pltpu.SemaphoreType.DMA,
            pltpu.SemaphoreType.DMA,
            pltpu.SemaphoreType.DMA,
            pltpu.SemaphoreType.DMA,
        ],
    )
    outs = k(emb2, node_p, src2, dst2, ea2, ean)
    return outs[6:10], outs[4]


def _k3_body(a0, a1, a2, a3, m0, m1, m2, m3, b1_ref, out_ref):
    y = jnp.dot(a0[...], m0[...], preferred_element_type=_f32)
    y += jnp.dot(a1[...], m1[...], preferred_element_type=_f32)
    y += jnp.dot(a2[...], m2[...], preferred_element_type=_f32)
    y += jnp.dot(a3[...], m3[...], preferred_element_type=_f32)
    out_ref[...] = jnp.maximum(y + b1_ref[...], 0.0)


def _run_k3(aggs_packed, Ms, b1t):
    blk = 448
    grid = (N_PAD // 4) // blk
    y = pl.pallas_call(
        _k3_body,
        grid=(grid,),
        in_specs=[pl.BlockSpec((blk, F), lambda i: (i, 0))] * 4
        + [pl.BlockSpec((F, 4 * F), lambda i: (0, 0))] * 4
        + [pl.BlockSpec((1, 4 * F), lambda i: (0, 0))],
        out_specs=pl.BlockSpec((blk, 4 * F), lambda i: (i, 0)),
        out_shape=jax.ShapeDtypeStruct((N_PAD // 4, 4 * F), _f32),
    )(*aggs_packed, *Ms, b1t)
    return y.reshape(N_PAD, F)


def _k4_body(h1_hbm, src2, dst2, w2, batch_p,
             out2, outc,
             sidx4, didx4, gv4, wv4, bufa, bufb, zbuf2, obuf, gvn, gvn_t,
             idv, accs, acc2, accc, semga, semgb, semsa, semsb, semi):
    core = lax.axis_index("c")
    s = lax.axis_index("s")
    wid = s * NC + core
    bufs = (bufa, bufb)
    semG = (semga, semgb)
    semS = (semsa, semsb)
    hdummy = h1_hbm.at[pl.ds(0, BLK)]

    def zr(k, carry):
        for j in range(F // 16):
            obuf[k, pl.ds(16 * j, 16)] = jnp.full((16,), 1.0, _f32)
        return carry

    lax.fori_loop(0, BLK, zr, 0)

    def zr2(k, carry):
        for j in range(F // 16):
            zbuf2[k, pl.ds(16 * j, 16)] = jnp.zeros((16,), _f32)
        return carry

    lax.fori_loop(0, 8, zr2, 0)

    slab = s * ACC2_ROWS

    def zr3(k, carry):
        pltpu.sync_copy(zbuf2, accs.at[pl.ds(slab + k * 8, 8)])
        return carry

    lax.fori_loop(0, ACC2_ROWS // 8, zr3, 0)
    base16 = lax.iota(_i32, 16)
    for q in range(NUM_GRAPHS // 16):
        idv[pl.ds(16 * q, 16)] = base16 + 16 * q

    @pl.when(s == 0)
    def _zero_acc():
        def zcp(k, carry):
            pltpu.sync_copy(zbuf2, acc2.at[pl.ds(k * 8, 8)])
            return carry

        lax.fori_loop(0, ACC2_ROWS // 8, zcp, 0)

        def zcc(k, carry):
            pltpu.sync_copy(zbuf2, accc.at[pl.ds(k * 8, 8)])
            return carry

        lax.fori_loop(0, ACC2_ROWS // 8, zcc, 0)

    plsc.subcore_barrier()

    rbase = wid * (EPW // BLK)

    def eb(i, carry):
        row = rbase + i * 4
        pltpu.sync_copy(src2.at[pl.ds(row, 4)], sidx4)
        pltpu.sync_copy(dst2.at[pl.ds(row, 4)], didx4)
        pltpu.sync_copy(w2.at[pl.ds(row, 4)], wv4)
        for k in range(4):
            pltpu.async_copy(batch_p.at[didx4.at[k]], gv4.at[k], semi)

        @pl.when(i > 0)
        def _drain0():
            pltpu.make_async_copy(hdummy, bufs[0], semS[0]).wait()

        pltpu.async_copy(h1_hbm.at[sidx4.at[0]], bufs[0], semG[0])
        pltpu.make_async_copy(src2.at[pl.ds(0, 4)], gv4, semi).wait()
        soff = jnp.full((16,), slab, _i32)
        for k in range(4):
            for v in range(BLK // 16):
                gv4[k, pl.ds(16 * v, 16)] = gv4[k, pl.ds(16 * v, 16)] + soff
        for k in range(4):
            if k < 3:
                nb = (k + 1) % 2
                if k == 0:
                    @pl.when(i > 0)
                    def _drain1():
                        pltpu.make_async_copy(hdummy, bufs[1],
                                              semS[1]).wait()
                else:
                    pltpu.make_async_copy(hdummy, bufs[nb], semS[nb]).wait()
                pltpu.async_copy(h1_hbm.at[sidx4.at[k + 1]], bufs[nb],
                                 semG[nb])
            pltpu.make_async_copy(hdummy, bufs[k % 2], semG[k % 2]).wait()

            def mg(g, c2, k=k):
                _mul_rows(bufs[k % 2], wv4, k, g, F // 16)
                return c2

            lax.fori_loop(0, 8, mg, 0)
            pltpu.async_copy(bufs[k % 2], accs.at[gv4.at[k]], semS[k % 2],
                             add=True)
        return carry

    lax.fori_loop(0, EPW // BLK // 4, eb, 0)
    pltpu.make_async_copy(hdummy, bufs[0], semS[0]).wait()
    pltpu.make_async_copy(hdummy, bufs[1], semS[1]).wait()

    pltpu.sync_copy(accs.at[pl.ds(slab, NUM_GRAPHS)],
                    bufa.at[pl.ds(0, NUM_GRAPHS)])
    pltpu.sync_copy(bufa.at[pl.ds(0, NUM_GRAPHS)], acc2.at[idv], add=True)

    nbase = wid * NPW

    def cb(i, carry):
        pltpu.sync_copy(batch_p.at[pl.ds(nbase + i * BLK, BLK)], gvn)
        pltpu.sync_copy(obuf, accc.at[gvn], add=True)
        return carry

    lax.fori_loop(0, NPW // BLK, cb, 0)
    tb = nbase + (NPW // BLK) * BLK
    pltpu.sync_copy(batch_p.at[pl.ds(tb, NPW % BLK)], gvn_t)
    pltpu.sync_copy(obuf.at[pl.ds(0, NPW % BLK)], accc.at[gvn_t], add=True)

    plsc.subcore_barrier()

    @pl.when(s == 0)
    def _writeback():
        pltpu.sync_copy(acc2.at[pl.ds(0, NUM_GRAPHS)], out2.at[core])
        pltpu.sync_copy(accc.at[pl.ds(0, NUM_GRAPHS)], outc.at[core])


def _run_k4(h1, src2, dst2, w2, batch_p):
    k = pl.kernel(
        _k4_body,
        out_type=[jax.ShapeDtypeStruct((NC, NUM_GRAPHS, F), _f32),
                  jax.ShapeDtypeStruct((NC, NUM_GRAPHS, F), _f32)],
        mesh=_mesh,
        compiler_params=_params,
        scratch_types=[
            pltpu.VMEM((4, BLK), _i32),
            pltpu.VMEM((4, BLK), _i32),
            pltpu.VMEM((4, BLK), _i32),
            pltpu.VMEM((4, BLK), _f32),
            pltpu.VMEM((BLK, F), _f32),
            pltpu.VMEM((BLK, F), _f32),
            pltpu.VMEM((8, F), _f32),
            pltpu.VMEM((BLK, F), _f32),
            pltpu.VMEM((BLK,), _i32),
            pltpu.VMEM((NPW % BLK,), _i32),
            pltpu.VMEM((NUM_GRAPHS,), _i32),
            pltpu.VMEM_SHARED((NS * ACC2_ROWS, F), _f32),
            pltpu.VMEM_SHARED((ACC2_ROWS, F), _f32),
            pltpu.VMEM_SHARED((ACC2_ROWS, F), _f32),
            pltpu.SemaphoreType.DMA,
            pltpu.SemaphoreType.DMA,
            pltpu.SemaphoreType.DMA,
            pltpu.SemaphoreType.DMA,
            pltpu.SemaphoreType.DMA,
        ],
    )
    return k(h1, src2, dst2, w2, batch_p)


def _k5_body(o2_ref, oc_ref, w2_ref, b2_ref, out_ref):
    summed = o2_ref[0] + o2_ref[1]
    cnt = oc_ref[0] + oc_ref[1]
    cnt1 = jnp.maximum(cnt[:, 0:1], 1.0)
    pooled = summed * (1.0 / cnt1)
    logits = (jnp.dot(pooled[:, :DIM], w2_ref[...],
                      preferred_element_type=_f32) + b2_ref[...])
    out_ref[...] = logits


def _run_k5(out2, outc, W2, b2):
    return pl.pallas_call(
        _k5_body,
        out_shape=jax.ShapeDtypeStruct((NUM_GRAPHS, NUM_CLASS), _f32),
    )(out2, outc, W2, b2.reshape(1, NUM_CLASS))


def kernel(node, adj, edge_attr, batch, embedding, ean, etans, W1, b1, W2,
           b2):
    node = node.astype(_i32)
    adj = adj.astype(_i32)
    edge_attr = edge_attr.astype(_i32)
    batch = batch.astype(_i32)

    epad = E_PAD - E
    npad = N_PAD - N
    src2 = jnp.concatenate([adj[0], jnp.zeros((epad,), _i32)]).reshape(
        E_PAD // BLK, BLK)
    dst2 = jnp.concatenate([adj[1], jnp.full((epad,), N, _i32)]).reshape(
        E_PAD // BLK, BLK)
    ea2 = jnp.concatenate([edge_attr, jnp.zeros((epad,), _i32)]).reshape(
        E_PAD // BLK, BLK)
    node_p = jnp.concatenate([node, jnp.zeros((npad,), _i32)])
    batch_p = jnp.concatenate([batch, jnp.full((npad,), NUM_GRAPHS, _i32)])
    W1p = jnp.pad(W1, ((0, F - GLOVE), (0, F - DIM)))
    eye4 = jnp.eye(4, dtype=_f32)
    Ms = [jnp.kron(eye4, W1p[32 * c:32 * c + 32, :]) for c in range(4)]
    b1t = jnp.tile(jnp.pad(b1, (0, F - DIM)), 4).reshape(1, 4 * F)

    emb2 = _run_k0(embedding, etans)
    aggs, w2a = _run_k2(emb2, node_p, src2, dst2, ea2, ean)
    aggs_packed = [a.reshape(N_PAD // 4, F) for a in aggs]
    h1 = _run_k3(aggs_packed, Ms, b1t)
    out2, outc = _run_k4(h1, src2, dst2, w2a, batch_p)
    return _run_k5(out2, outc, W2, b2)


# speedup vs baseline: 1.1089x; 1.0213x over previous
import jax
import jax.numpy as jnp
from jax import lax
from jax.experimental import pallas as pl
from jax.experimental.pallas import tpu as pltpu
from jax.experimental.pallas import tpu_sc as plsc

NODE_NUM = 100000
N = 50000
E = 800000
NUM_GRAPHS = 64
GLOVE = 100
DIM = 100
NUM_CLASS = 52

NC, NS, LANES = 2, 16, 16
NW = NC * NS
N_PAD = 50176
E_PAD = 802816
NPW = N_PAD // NW
EPW = E_PAD // NW
EPT = E_PAD // NS
BLK = 128
F = 128
CH = 32
ACC2_ROWS = 72

_mesh = plsc.VectorSubcoreMesh(
    core_axis_name="c", subcore_axis_name="s", num_cores=NC, num_subcores=NS)
_params = pltpu.CompilerParams(
    use_tc_tiling_on_sc=False, needs_layout_passes=False)

_f32 = jnp.float32
_i32 = jnp.int32


def _k0_body(emb_ref, et_ref, out_ref):
    x = emb_ref[...] * et_ref[...]
    out_ref[...] = jnp.concatenate(
        [x, jnp.zeros((x.shape[0], F - GLOVE), _f32)], axis=1)


def _run_k0(embedding, etans):
    blk = 1000
    return pl.pallas_call(
        _k0_body,
        grid=(NODE_NUM // blk,),
        in_specs=[
            pl.BlockSpec((blk, GLOVE), lambda i: (i, 0)),
            pl.BlockSpec((blk, 1), lambda i: (i, 0)),
        ],
        out_specs=pl.BlockSpec((blk, F), lambda i: (i, 0)),
        out_shape=jax.ShapeDtypeStruct((NODE_NUM, F), _f32),
    )(embedding, etans.reshape(NODE_NUM, 1))


def _mul_rows(buf, wref, krow, g, nv):
    w16 = wref[krow, pl.ds(g * 16, 16)]
    dn = lax.GatherDimensionNumbers(
        offset_dims=(), collapsed_slice_dims=(0,), start_index_map=(0,))
    for j in range(16):
        e = g * 16 + j
        ws = lax.gather(w16, jnp.full((16, 1), j, _i32), dn, (1,),
                        mode=lax.GatherScatterMode.PROMISE_IN_BOUNDS)
        for v in range(nv):
            buf[e, pl.ds(16 * v, 16)] = buf[e, pl.ds(16 * v, 16)] * ws


def _k2_body(emb2, node_p, src2, dst2, ea2, ean,
             h0, h1, h2, h3, w0, w1, agg0, agg1, agg2, agg3,
             sidx4, didx4, wv4, bufa, bufb, cbuf, cbt, zbuf, acc,
             semga, semgb, semsa, semsb):
    core = lax.axis_index("c")
    s = lax.axis_index("s")
    hrefs = (h0, h1, h2, h3)
    arefs = (agg0, agg1, agg2, agg3)
    wrefs = (w0, w1)
    bufs = (bufa, bufb)
    semG = (semga, semgb)
    semS = (semsa, semsb)

    def zr(k, carry):
        zbuf[k, pl.ds(0, 16)] = jnp.zeros((16,), _f32)
        zbuf[k, pl.ds(16, 16)] = jnp.zeros((16,), _f32)
        return carry

    lax.fori_loop(0, 64, zr, 0)
    zoff = s * (N_PAD // NS)
    rbase = s * (EPT // BLK)
    NPT = N_PAD // NS

    for c_id in range(NC):
        @pl.when(core == c_id)
        def _phase0(c_id=c_id):
            def node_block(base, nb):
                pltpu.sync_copy(node_p.at[pl.ds(base, nb)],
                                sidx4.at[0].at[pl.ds(0, nb)])
                pltpu.async_copy(
                    emb2.at[sidx4.at[0].at[pl.ds(0, nb)]],
                    cbuf.at[pl.ds(0, nb)], semG[0]).wait()
                for t in range(2):
                    chunk = 2 * c_id + t

                    def cp(e, carry, chunk=chunk):
                        cbt[e, pl.ds(0, 16)] = \
                            cbuf[e, pl.ds(32 * chunk, 16)]
                        cbt[e, pl.ds(16, 16)] = \
                            cbuf[e, pl.ds(32 * chunk + 16, 16)]
                        return carry

                    lax.fori_loop(0, nb, cp, 0)
                    pltpu.sync_copy(cbt.at[pl.ds(0, nb)],
                                    hrefs[chunk].at[pl.ds(base, nb)])

            def blockA(i, carry):
                node_block(s * NPT + i * 64, 64)
                return carry

            lax.fori_loop(0, NPT // 64, blockA, 0)

            def blockW(i, carry):
                row = rbase + i * 4
                pltpu.sync_copy(ea2.at[pl.ds(row, 4)], didx4)
                descs = [pltpu.async_copy(ean.at[didx4.at[k]],
                                          wv4.at[k], semG[0])
                         for k in range(4)]
                for d in descs:
                    d.wait()
                pltpu.sync_copy(wv4, wrefs[c_id].at[pl.ds(row, 4)])
                return carry

            lax.fori_loop(0, EPT // BLK // 4, blockW, 0)

    for ci in range(2):
        def zcp(k, carry):
            pltpu.sync_copy(zbuf, acc.at[pl.ds(zoff + k * 64, 64)])
            return carry

        lax.fori_loop(0, (N_PAD // NS) // 64, zcp, 0)
        plsc.subcore_barrier()

        for c_id in range(NC):
            chunk = c_id * 2 + ci

            @pl.when(core == c_id)
            def _scatter(chunk=chunk, c_id=c_id):
                hdummy = hrefs[chunk].at[pl.ds(0, BLK)]

                def sb(i, carry):
                    row = rbase + i * 4
                    pltpu.sync_copy(src2.at[pl.ds(row, 4)], sidx4)
                    pltpu.sync_copy(dst2.at[pl.ds(row, 4)], didx4)
                    pltpu.sync_copy(wrefs[c_id].at[pl.ds(row, 4)], wv4)

                    @pl.when(i > 0)
                    def _drain0():
                        pltpu.make_async_copy(hdummy, bufs[0],
                                              semS[0]).wait()

                    pltpu.async_copy(hrefs[chunk].at[sidx4.at[0]],
                                     bufs[0], semG[0])
                    for k in range(4):
                        if k < 3:
                            nb = (k + 1) % 2
                            if k == 0:
                                @pl.when(i > 0)
                                def _drain1():
                                    pltpu.make_async_copy(
                                        hdummy, bufs[1], semS[1]).wait()
                            else:
                                pltpu.make_async_copy(
                                    hdummy, bufs[nb], semS[nb]).wait()
                            pltpu.async_copy(
                                hrefs[chunk].at[sidx4.at[k + 1]],
                                bufs[nb], semG[nb])
                        pltpu.make_async_copy(hdummy, bufs[k % 2],
                                              semG[k % 2]).wait()

                        def mg(g, c2, k=k):
                            _mul_rows(bufs[k % 2], wv4, k, g, 2)
                            return c2

                        lax.fori_loop(0, 8, mg, 0)
                        pltpu.async_copy(bufs[k % 2], acc.at[didx4.at[k]],
                                         semS[k % 2], add=True)
                    return carry

                lax.fori_loop(0, EPT // BLK // 4, sb, 0)
                pltpu.make_async_copy(hdummy, bufs[0], semS[0]).wait()
                pltpu.make_async_copy(hdummy, bufs[1], semS[1]).wait()

        plsc.subcore_barrier()

        for c_id in range(NC):
            chunk = c_id * 2 + ci

            @pl.when(core == c_id)
            def _writeback(chunk=chunk):
                def wb(k, carry):
                    off = zoff + k * 64
                    pltpu.sync_copy(acc.at[pl.ds(off, 64)],
                                    arefs[chunk].at[pl.ds(off, 64)])
                    return carry

                lax.fori_loop(0, (N_PAD // NS) // 64, wb, 0)

        plsc.subcore_barrier()


def _run_k2(emb2, node_p, src2, dst2, ea2, ean):
    out_type = ([jax.ShapeDtypeStruct((N_PAD, CH), _f32)] * 4
                + [jax.ShapeDtypeStruct((E_PAD // BLK, BLK), _f32)] * 2
                + [jax.ShapeDtypeStruct((N_PAD, CH), _f32)] * 4)
    k = pl.kernel(
        _k2_body,
        out_type=out_type,
        mesh=_mesh,
        compiler_params=_params,
        scratch_types=[
            pltpu.VMEM((4, BLK), _i32),
            pltpu.VMEM((4, BLK), _i32),
            pltpu.VMEM((4, BLK), _f32),
            pltpu.VMEM((BLK, CH), _f32),
            pltpu.VMEM((BLK, CH), _f32),
            pltpu.VMEM((64, F), _f32),
            pltpu.VMEM((64, CH), _f32),
            pltpu.VMEM((64, CH), _f32),
            pltpu.VMEM_SHARED((N_PAD, CH), _f32),
            pltpu.SemaphoreType.DMA,
            pltpu.SemaphoreType.DMA,
            pltpu.SemaphoreType.DMA,
            pltpu.SemaphoreType.DMA,
        ],
    )
    outs = k(emb2, node_p, src2, dst2, ea2, ean)
    return outs[6:10], outs[4]


def _k3_body(a0, a1, a2, a3, m0, m1, m2, m3, b1_ref, out_ref):
    y = jnp.dot(a0[...], m0[...], preferred_element_type=_f32)
    y += jnp.dot(a1[...], m1[...], preferred_element_type=_f32)
    y += jnp.dot(a2[...], m2[...], preferred_element_type=_f32)
    y += jnp.dot(a3[...], m3[...], preferred_element_type=_f32)
    y = jnp.maximum(y + b1_ref[...], 0.0)
    out_ref[...] = y.reshape(y.shape[0] * 4, F)


def _run_k3(aggs_packed, Ms, b1t):
    blk = 448
    grid = (N_PAD // 4) // blk
    return pl.pallas_call(
        _k3_body,
        grid=(grid,),
        in_specs=[pl.BlockSpec((blk, F), lambda i: (i, 0))] * 4
        + [pl.BlockSpec((F, 4 * F), lambda i: (0, 0))] * 4
        + [pl.BlockSpec((1, 4 * F), lambda i: (0, 0))],
        out_specs=pl.BlockSpec((4 * blk, F), lambda i: (i, 0)),
        out_shape=jax.ShapeDtypeStruct((N_PAD, F), _f32),
    )(*aggs_packed, *Ms, b1t)


def _k4_body(h1_hbm, src2, dst2, w2, batch_p,
             out2, outc,
             sidx4, didx4, gv4, wv4, bufa, bufb, zbuf2, obuf, gvn, gvn_t,
             idv, accs, acc2, accc, semga, semgb, semsa, semsb, semi):
    core = lax.axis_index("c")
    s = lax.axis_index("s")
    wid = s * NC + core
    bufs = (bufa, bufb)
    semG = (semga, semgb)
    semS = (semsa, semsb)
    hdummy = h1_hbm.at[pl.ds(0, BLK)]

    def zr(k, carry):
        for j in range(F // 16):
            obuf[k, pl.ds(16 * j, 16)] = jnp.full((16,), 1.0, _f32)
        return carry

    lax.fori_loop(0, BLK, zr, 0)

    def zr2(k, carry):
        for j in range(F // 16):
            zbuf2[k, pl.ds(16 * j, 16)] = jnp.zeros((16,), _f32)
        return carry

    lax.fori_loop(0, 8, zr2, 0)

    slab = s * ACC2_ROWS

    def zr3(k, carry):
        pltpu.sync_copy(zbuf2, accs.at[pl.ds(slab + k * 8, 8)])
        return carry

    lax.fori_loop(0, ACC2_ROWS // 8, zr3, 0)
    base16 = lax.iota(_i32, 16)
    for q in range(NUM_GRAPHS // 16):
        idv[pl.ds(16 * q, 16)] = base16 + 16 * q

    @pl.when(s == 0)
    def _zero_acc():
        def zcp(k, carry):
            pltpu.sync_copy(zbuf2, acc2.at[pl.ds(k * 8, 8)])
            return carry

        lax.fori_loop(0, ACC2_ROWS // 8, zcp, 0)

        def zcc(k, carry):
            pltpu.sync_copy(zbuf2, accc.at[pl.ds(k * 8, 8)])
            return carry

        lax.fori_loop(0, ACC2_ROWS // 8, zcc, 0)

    plsc.subcore_barrier()

    rbase = wid * (EPW // BLK)

    def eb(i, carry):
        row = rbase + i * 4
        pltpu.sync_copy(src2.at[pl.ds(row, 4)], sidx4)
        pltpu.sync_copy(dst2.at[pl.ds(row, 4)], didx4)
        pltpu.sync_copy(w2.at[pl.ds(row, 4)], wv4)
        for k in range(4):
            pltpu.async_copy(batch_p.at[didx4.at[k]], gv4.at[k], semi)

        @pl.when(i > 0)
        def _drain0():
            pltpu.make_async_copy(hdummy, bufs[0], semS[0]).wait()

        pltpu.async_copy(h1_hbm.at[sidx4.at[0]], bufs[0], semG[0])
        pltpu.make_async_copy(src2.at[pl.ds(0, 4)], gv4, semi).wait()
        soff = jnp.full((16,), slab, _i32)
        for k in range(4):
            for v in range(BLK // 16):
                gv4[k, pl.ds(16 * v, 16)] = gv4[k, pl.ds(16 * v, 16)] + soff
        for k in range(4):
            if k < 3:
                nb = (k + 1) % 2
                if k == 0:
                    @pl.when(i > 0)
                    def _drain1():
                        pltpu.make_async_copy(hdummy, bufs[1],
                                              semS[1]).wait()
                else:
                    pltpu.make_async_copy(hdummy, bufs[nb], semS[nb]).wait()
                pltpu.async_copy(h1_hbm.at[sidx4.at[k + 1]], bufs[nb],
                                 semG[nb])
            pltpu.make_async_copy(hdummy, bufs[k % 2], semG[k % 2]).wait()

            def mg(g, c2, k=k):
                _mul_rows(bufs[k % 2], wv4, k, g, F // 16)
                return c2

            lax.fori_loop(0, 8, mg, 0)
            pltpu.async_copy(bufs[k % 2], accs.at[gv4.at[k]], semS[k % 2],
                             add=True)
        return carry

    lax.fori_loop(0, EPW // BLK // 4, eb, 0)
    pltpu.make_async_copy(hdummy, bufs[0], semS[0]).wait()
    pltpu.make_async_copy(hdummy, bufs[1], semS[1]).wait()

    pltpu.sync_copy(accs.at[pl.ds(slab, NUM_GRAPHS)],
                    bufa.at[pl.ds(0, NUM_GRAPHS)])
    pltpu.sync_copy(bufa.at[pl.ds(0, NUM_GRAPHS)], acc2.at[idv], add=True)

    nbase = wid * NPW

    def cb(i, carry):
        pltpu.sync_copy(batch_p.at[pl.ds(nbase + i * BLK, BLK)], gvn)
        pltpu.sync_copy(obuf, accc.at[gvn], add=True)
        return carry

    lax.fori_loop(0, NPW // BLK, cb, 0)
    tb = nbase + (NPW // BLK) * BLK
    pltpu.sync_copy(batch_p.at[pl.ds(tb, NPW % BLK)], gvn_t)
    pltpu.sync_copy(obuf.at[pl.ds(0, NPW % BLK)], accc.at[gvn_t], add=True)

    plsc.subcore_barrier()

    @pl.when(s == 0)
    def _writeback():
        pltpu.sync_copy(acc2.at[pl.ds(0, NUM_GRAPHS)], out2.at[core])
        pltpu.sync_copy(accc.at[pl.ds(0, NUM_GRAPHS)], outc.at[core])


def _run_k4(h1, src2, dst2, w2, batch_p):
    k = pl.kernel(
        _k4_body,
        out_type=[jax.ShapeDtypeStruct((NC, NUM_GRAPHS, F), _f32),
                  jax.ShapeDtypeStruct((NC, NUM_GRAPHS, F), _f32)],
        mesh=_mesh,
        compiler_params=_params,
        scratch_types=[
            pltpu.VMEM((4, BLK), _i32),
            pltpu.VMEM((4, BLK), _i32),
            pltpu.VMEM((4, BLK), _i32),
            pltpu.VMEM((4, BLK), _f32),
            pltpu.VMEM((BLK, F), _f32),
            pltpu.VMEM((BLK, F), _f32),
            pltpu.VMEM((8, F), _f32),
            pltpu.VMEM((BLK, F), _f32),
            pltpu.VMEM((BLK,), _i32),
            pltpu.VMEM((NPW % BLK,), _i32),
            pltpu.VMEM((NUM_GRAPHS,), _i32),
            pltpu.VMEM_SHARED((NS * ACC2_ROWS, F), _f32),
            pltpu.VMEM_SHARED((ACC2_ROWS, F), _f32),
            pltpu.VMEM_SHARED((ACC2_ROWS, F), _f32),
            pltpu.SemaphoreType.DMA,
            pltpu.SemaphoreType.DMA,
            pltpu.SemaphoreType.DMA,
            pltpu.SemaphoreType.DMA,
            pltpu.SemaphoreType.DMA,
        ],
    )
    return k(h1, src2, dst2, w2, batch_p)


def _k5_body(o2_ref, oc_ref, w2_ref, b2_ref, out_ref):
    summed = o2_ref[0] + o2_ref[1]
    cnt = oc_ref[0] + oc_ref[1]
    cnt1 = jnp.maximum(cnt[:, 0:1], 1.0)
    pooled = summed * (1.0 / cnt1)
    logits = (jnp.dot(pooled[:, :DIM], w2_ref[...],
                      preferred_element_type=_f32) + b2_ref[...])
    out_ref[...] = logits


def _run_k5(out2, outc, W2, b2):
    return pl.pallas_call(
        _k5_body,
        out_shape=jax.ShapeDtypeStruct((NUM_GRAPHS, NUM_CLASS), _f32),
    )(out2, outc, W2, b2.reshape(1, NUM_CLASS))


def kernel(node, adj, edge_attr, batch, embedding, ean, etans, W1, b1, W2,
           b2):
    node = node.astype(_i32)
    adj = adj.astype(_i32)
    edge_attr = edge_attr.astype(_i32)
    batch = batch.astype(_i32)

    epad = E_PAD - E
    npad = N_PAD - N
    src2 = jnp.concatenate([adj[0], jnp.zeros((epad,), _i32)]).reshape(
        E_PAD // BLK, BLK)
    dst2 = jnp.concatenate([adj[1], jnp.full((epad,), N, _i32)]).reshape(
        E_PAD // BLK, BLK)
    ea2 = jnp.concatenate([edge_attr, jnp.zeros((epad,), _i32)]).reshape(
        E_PAD // BLK, BLK)
    node_p = jnp.concatenate([node, jnp.zeros((npad,), _i32)])
    batch_p = jnp.concatenate([batch, jnp.full((npad,), NUM_GRAPHS, _i32)])
    W1p = jnp.pad(W1, ((0, F - GLOVE), (0, F - DIM)))
    eye4 = jnp.eye(4, dtype=_f32)
    Ms = [jnp.kron(eye4, W1p[32 * c:32 * c + 32, :]) for c in range(4)]
    b1t = jnp.tile(jnp.pad(b1, (0, F - DIM)), 4).reshape(1, 4 * F)

    emb2 = _run_k0(embedding, etans)
    aggs, w2a = _run_k2(emb2, node_p, src2, dst2, ea2, ean)
    aggs_packed = [a.reshape(N_PAD // 4, F) for a in aggs]
    h1 = _run_k3(aggs_packed, Ms, b1t)
    out2, outc = _run_k4(h1, src2, dst2, w2a, batch_p)
    return _run_k5(out2, outc, W2, b2)
